# Initial kernel scaffold; baseline (speedup 1.0000x reference)
import jax
import jax.numpy as jnp
from jax.experimental import pallas as pl


def kernel(x, batch, edge_index, row, col, W1, b1, W2, b2, W3, b3, fc_w, fc_b, lin1_w, lin1_b, lin2_w, lin2_b):
    raise NotImplementedError("write your pallas kernel here")


# speedup vs baseline: 51.7771x; 51.7771x over previous
import functools

import jax
import jax.numpy as jnp
from jax import lax
from jax.experimental import pallas as pl
from jax.experimental.pallas import tpu as pltpu
from jax.experimental.pallas import tpu_sc as plsc

_f32 = jnp.float32

_N = 15135
_BS = 4
_H = 32
_NP = 15360
_E = 242160
_P = 150000
_NCMT = 1387
_CP = 1408

_NC, _NS = 2, 16
_C = 128
_ECH = -(-_E // (_NC * _NS * _C))
_EW = _ECH * _C
_EP = _EW * _NC * _NS
_PCH = -(-_P // (_NC * _NS * _C))
_PW = _PCH * _C
_PP = _PW * _NC * _NS
_RZ = _NP // _NS
_CZ = _CP // _NS

_mesh = plsc.VectorSubcoreMesh(core_axis_name="c", subcore_axis_name="s")


@functools.partial(
    pl.kernel, mesh=_mesh,
    compiler_params=pltpu.CompilerParams(use_tc_tiling_on_sc=False),
    out_type=jax.ShapeDtypeStruct((_NC, _NS, _RZ), _f32),
    scratch_types=[
        pltpu.VMEM((_C,), jnp.int32),
        pltpu.VMEM((_C,), _f32),
        pltpu.VMEM((_RZ,), _f32),
        pltpu.VMEM_SHARED((_NP,), _f32),
        pltpu.SemaphoreType.DMA,
    ],
)
def _deg_sc(dstp, zflat, oflat, out, idx_d, ones_v, zb1, accd, sem):
    c = lax.axis_index("c")
    s = lax.axis_index("s")
    w = c * _NS + s
    pltpu.sync_copy(zflat, zb1)
    pltpu.sync_copy(oflat, ones_v)
    pltpu.sync_copy(zb1, accd.at[pl.ds(s * _RZ, _RZ)])
    plsc.subcore_barrier()

    @pl.loop(0, _ECH)
    def _(j):
        b = w * _EW + j * _C
        pltpu.sync_copy(dstp.at[pl.ds(b, _C)], idx_d)
        pltpu.sync_copy(ones_v, accd.at[idx_d], add=True)

    plsc.subcore_barrier()
    pltpu.sync_copy(accd.at[pl.ds(s * _RZ, _RZ)], zb1)
    pltpu.sync_copy(zb1, out.at[c, s])


@functools.partial(
    pl.kernel, mesh=_mesh,
    compiler_params=pltpu.CompilerParams(use_tc_tiling_on_sc=False),
    out_type=[jax.ShapeDtypeStruct((_NC, _NS, _RZ, 64), _f32),
              jax.ShapeDtypeStruct((_NC, _NS, _RZ, 64), _f32)],
    scratch_types=[
        pltpu.VMEM((_C,), jnp.int32),
        pltpu.VMEM((_C,), jnp.int32),
        pltpu.VMEM((_C, 64), _f32),
        pltpu.VMEM((32, 64), _f32),
        pltpu.VMEM_SHARED((_NP, 64), _f32),
        pltpu.SemaphoreType.DMA,
    ],
)
def _spmm_sc(ta, tb, srcp, dstp, zsmall, outa, outb,
             idx_s, idx_d, rows, zb, acc, sem):
    c = lax.axis_index("c")
    s = lax.axis_index("s")
    w = c * _NS + s
    pltpu.sync_copy(zsmall, zb)

    for table, out in ((ta, outa), (tb, outb)):
        @pl.loop(0, _RZ // 32)
        def _(i):
            pltpu.sync_copy(zb, acc.at[pl.ds(s * _RZ + i * 32, 32)])

        plsc.subcore_barrier()

        @pl.loop(0, _ECH)
        def _(j):
            b = w * _EW + j * _C
            pltpu.sync_copy(srcp.at[pl.ds(b, _C)], idx_s)
            pltpu.sync_copy(dstp.at[pl.ds(b, _C)], idx_d)
            pltpu.async_copy(table.at[idx_s], rows, sem).wait()
            pltpu.sync_copy(rows, acc.at[idx_d], add=True)

        plsc.subcore_barrier()

        @pl.loop(0, _RZ // 32)
        def _(i):
            pltpu.sync_copy(acc.at[pl.ds(s * _RZ + i * 32, 32)], zb)
            pltpu.sync_copy(zb, out.at[c, s, pl.ds(i * 32, 32)])


@functools.partial(
    pl.kernel, mesh=_mesh,
    compiler_params=pltpu.CompilerParams(use_tc_tiling_on_sc=False),
    out_type=jax.ShapeDtypeStruct((_NC, _NS, _CZ, 8), _f32),
    scratch_types=[
        pltpu.VMEM((_C,), jnp.int32),
        pltpu.VMEM((_C,), jnp.int32),
        pltpu.VMEM((_C, 8), _f32),
        pltpu.VMEM((_CZ, 8), _f32),
        pltpu.VMEM_SHARED((_CP, 8), _f32),
        pltpu.SemaphoreType.DMA,
    ],
)
def _pool_sc(y, rowp, colp, zpool, out, idx_r, idx_c, rows, zbp, accp, sem):
    c = lax.axis_index("c")
    s = lax.axis_index("s")
    w = c * _NS + s
    pltpu.sync_copy(zpool, zbp)
    pltpu.sync_copy(zbp, accp.at[pl.ds(s * _CZ, _CZ)])
    plsc.subcore_barrier()

    @pl.loop(0, _PCH)
    def _(j):
        b = w * _PW + j * _C
        pltpu.sync_copy(rowp.at[pl.ds(b, _C)], idx_r)
        pltpu.sync_copy(colp.at[pl.ds(b, _C)], idx_c)
        pltpu.async_copy(y.at[idx_r], rows, sem).wait()
        pltpu.sync_copy(rows, accp.at[idx_c], add=True)

    plsc.subcore_barrier()
    pltpu.sync_copy(accp.at[pl.ds(s * _CZ, _CZ)], zbp)
    pltpu.sync_copy(zbp, out.at[c, s])


def _prep1(xtp, w1b, dega, degb):
    def body(x_ref, w_ref, da, db, o_ref):
        inv = lax.rsqrt(1.0 + da[...] + db[...])
        o_ref[...] = inv * jnp.dot(x_ref[...], w_ref[...],
                                   preferred_element_type=_f32)

    return pl.pallas_call(
        body,
        grid=(_NP // 128,),
        in_specs=[
            pl.BlockSpec((128, _BS * 16), lambda i: (i, 0)),
            pl.BlockSpec((_BS * 16, 128), lambda i: (0, 0)),
            pl.BlockSpec((128, 1), lambda i: (i, 0)),
            pl.BlockSpec((128, 1), lambda i: (i, 0)),
        ],
        out_specs=pl.BlockSpec((128, 128), lambda i: (i, 0)),
        out_shape=jax.ShapeDtypeStruct((_NP, 128), _f32),
    )(xtp, w1b, dega, degb)


def _prep23(acca, accb, msgp, dega, degb, wbig, btile):
    def body(aa, ab, mp, da, db, w_ref, b_ref, o_msg, o_xr):
        inv = lax.rsqrt(1.0 + da[...] + db[...])
        xr = jnp.maximum(inv * (aa[...] + ab[...] + mp[...]) + b_ref[...], 0.0)
        o_xr[...] = xr
        o_msg[...] = inv * jnp.dot(xr, w_ref[...], preferred_element_type=_f32)

    blk = pl.BlockSpec((128, 128), lambda i: (i, 0))
    return pl.pallas_call(
        body,
        grid=(_NP // 128,),
        in_specs=[
            blk, blk, blk,
            pl.BlockSpec((128, 1), lambda i: (i, 0)),
            pl.BlockSpec((128, 1), lambda i: (i, 0)),
            pl.BlockSpec((128, 128), lambda i: (0, 0)),
            pl.BlockSpec((1, 128), lambda i: (0, 0)),
        ],
        out_specs=[blk, blk],
        out_shape=[jax.ShapeDtypeStruct((_NP, 128), _f32),
                   jax.ShapeDtypeStruct((_NP, 128), _f32)],
    )(acca, accb, msgp, dega, degb, wbig, btile)


def _ykern(acca, accb, msg3, dega, degb, b3t, xr1, xr2, f1, f2, f3):
    def body(aa, ab, m3, da, db, b3, x1, x2, f1r, f2r, f3r, o_ref):
        i = pl.program_id(0)
        inv = lax.rsqrt(1.0 + da[...] + db[...])
        xr3 = jnp.maximum(inv * (aa[...] + ab[...] + m3[...]) + b3[...], 0.0)
        y = (jnp.dot(x1[...], f1r[...], preferred_element_type=_f32)
             + jnp.dot(x2[...], f2r[...], preferred_element_type=_f32)
             + jnp.dot(xr3, f3r[...], preferred_element_type=_f32))
        rowid = i * 128 + lax.broadcasted_iota(jnp.int32, (128, 8), 0)
        colid = lax.broadcasted_iota(jnp.int32, (128, 8), 1)
        o_ref[...] = y + jnp.where((rowid < _N) & (colid == 4), 1.0, 0.0)

    blk = pl.BlockSpec((128, 128), lambda i: (i, 0))
    fblk = pl.BlockSpec((128, 8), lambda i: (0, 0))
    return pl.pallas_call(
        body,
        grid=(_NP // 128,),
        in_specs=[
            blk, blk, blk,
            pl.BlockSpec((128, 1), lambda i: (i, 0)),
            pl.BlockSpec((128, 1), lambda i: (i, 0)),
            pl.BlockSpec((1, 128), lambda i: (0, 0)),
            blk, blk, fblk, fblk, fblk,
        ],
        out_specs=pl.BlockSpec((128, 8), lambda i: (i, 0)),
        out_shape=jax.ShapeDtypeStruct((_NP, 8), _f32),
    )(acca, accb, msg3, dega, degb, b3t, xr1, xr2, f1, f2, f3)


def _head(pa, pb, lin1p, l1b, l2w, l2b, fcb):
    def body(pa_r, pb_r, w1_r, b1_r, w2_r, b2_r, fb_r, o_ref):
        ps = pa_r[...] + pb_r[...]
        colid = lax.broadcasted_iota(jnp.int32, (_CP, 8), 1)
        is_cnt = (colid == 4).astype(_f32)
        cnt = jnp.maximum(jnp.sum(ps * is_cnt, axis=1, keepdims=True), 1.0)
        h = ps / cnt + fb_r[0, 0]
        z = lax.dot_general(h, w1_r[...], (((0,), (0,)), ((), ())),
                            preferred_element_type=_f32)
        z = jnp.maximum(z[0:_BS, :] + b1_r[...], 0.0)
        lg = jnp.dot(z, w2_r[...], preferred_element_type=_f32) + b2_r[...]
        m = jnp.max(lg, axis=1, keepdims=True)
        e = jnp.exp(lg - m)
        o_ref[...] = (lg - m) - jnp.log(jnp.sum(e, axis=1, keepdims=True))

    return pl.pallas_call(
        body,
        out_shape=jax.ShapeDtypeStruct((_BS, 2), _f32),
    )(pa, pb, lin1p, l1b, l2w, l2b, fcb)


def kernel(x, batch, edge_index, row, col, W1, b1, W2, b2, W3, b3,
           fc_w, fc_b, lin1_w, lin1_b, lin2_w, lin2_b):
    eye = jnp.eye(_BS, dtype=_f32)

    xt = jnp.transpose(x, (1, 0, 2)).reshape(_N, _BS * 16)
    xtp = jnp.pad(xt, ((0, _NP - _N), (0, 0)))

    srcp = jnp.pad(edge_index[0], (0, _EP - _E), constant_values=_N)
    dstp = jnp.pad(edge_index[1], (0, _EP - _E), constant_values=_N)
    rowp = jnp.pad(row, (0, _PP - _P), constant_values=_N)
    colp = jnp.pad(col, (0, _PP - _P), constant_values=_NCMT)

    w1b = jnp.kron(eye, W1)
    w2b = jnp.kron(eye, W2)
    w3b = jnp.kron(eye, W3)
    b1t = jnp.tile(b1, _BS)[None, :]
    b2t = jnp.tile(b2, _BS)[None, :]
    b3t = jnp.tile(b3, _BS)[None, :]
    fc3 = fc_w[:, 0].reshape(_H, 3)
    f1, f2, f3 = (jnp.pad(jnp.kron(eye, fc3[:, l:l + 1]), ((0, 0), (0, 4)))
                  for l in range(3))
    lin1p = jnp.pad(lin1_w, ((0, _CP - _NCMT), (0, 0)))

    zsmall = jnp.zeros((32, 64), _f32)
    zflat = jnp.zeros((_RZ,), _f32)
    oflat = jnp.ones((_C,), _f32)
    zpool = jnp.zeros((_CZ, 8), _f32)

    deg = _deg_sc(dstp, zflat, oflat)
    dega = deg[0].reshape(_NP, 1)
    degb = deg[1].reshape(_NP, 1)

    def spmm(msg):
        oa, ob = _spmm_sc(msg[:, :64], msg[:, 64:], srcp, dstp, zsmall)
        return [jnp.concatenate([oa[k].reshape(_NP, 64),
                                 ob[k].reshape(_NP, 64)], axis=1)
                for k in range(_NC)]

    msg1 = _prep1(xtp, w1b, dega, degb)
    a1 = spmm(msg1)
    msg2, xr1 = _prep23(a1[0], a1[1], msg1, dega, degb, w2b, b1t)
    a2 = spmm(msg2)
    msg3, xr2 = _prep23(a2[0], a2[1], msg2, dega, degb, w3b, b2t)
    a3 = spmm(msg3)
    y = _ykern(a3[0], a3[1], msg3, dega, degb, b3t, xr1, xr2,
               f1, f2, f3)
    pool = _pool_sc(y, rowp, colp, zpool)
    return _head(pool[0].reshape(_CP, 8), pool[1].reshape(_CP, 8),
                 lin1p, lin1_b[None, :], lin2_w, lin2_b[None, :],
                 fc_b.reshape(1, 1))


# speedup vs baseline: 66.1183x; 1.2770x over previous
import functools

import jax
import jax.numpy as jnp
from jax import lax
from jax.experimental import pallas as pl
from jax.experimental.pallas import tpu as pltpu
from jax.experimental.pallas import tpu_sc as plsc

_f32 = jnp.float32

_N = 15135
_BS = 4
_H = 32
_NP = 15360
_E = 242160
_P = 150000
_NCMT = 1387
_CP = 1408

_NC, _NS = 2, 16
_C = 128
_ECH = -(-_E // (_NC * _NS * _C))
_EW = _ECH * _C
_EP = _EW * _NC * _NS
_PCH = -(-_P // (_NC * _NS * _C))
_PW = _PCH * _C
_PP = _PW * _NC * _NS
_RZ = _NP // _NS
_CZ = _CP // _NS

_mesh = plsc.VectorSubcoreMesh(core_axis_name="c", subcore_axis_name="s")


@functools.partial(
    pl.kernel, mesh=_mesh,
    compiler_params=pltpu.CompilerParams(use_tc_tiling_on_sc=False),
    out_type=jax.ShapeDtypeStruct((_NC, _NS, _RZ), _f32),
    scratch_types=[
        pltpu.VMEM((_C,), jnp.int32),
        pltpu.VMEM((_C,), _f32),
        pltpu.VMEM((_RZ,), _f32),
        pltpu.VMEM_SHARED((_NP,), _f32),
        pltpu.SemaphoreType.DMA,
    ],
)
def _deg_sc(dstp, zflat, oflat, out, idx_d, ones_v, zb1, accd, sem):
    c = lax.axis_index("c")
    s = lax.axis_index("s")
    w = c * _NS + s
    pltpu.sync_copy(zflat, zb1)
    pltpu.sync_copy(oflat, ones_v)
    pltpu.sync_copy(zb1, accd.at[pl.ds(s * _RZ, _RZ)])
    plsc.subcore_barrier()

    @pl.loop(0, _ECH)
    def _(j):
        b = w * _EW + j * _C
        pltpu.sync_copy(dstp.at[pl.ds(b, _C)], idx_d)
        pltpu.sync_copy(ones_v, accd.at[idx_d], add=True)

    plsc.subcore_barrier()
    pltpu.sync_copy(accd.at[pl.ds(s * _RZ, _RZ)], out.at[c, s])


@functools.partial(
    pl.kernel, mesh=_mesh,
    compiler_params=pltpu.CompilerParams(use_tc_tiling_on_sc=False),
    out_type=[jax.ShapeDtypeStruct((_NC, _NS, _RZ, 64), _f32),
              jax.ShapeDtypeStruct((_NC, _NS, _RZ, 64), _f32)],
    scratch_types=[
        pltpu.VMEM((_C,), jnp.int32),
        pltpu.VMEM((_C,), jnp.int32),
        pltpu.VMEM((_C,), jnp.int32),
        pltpu.VMEM((_C,), jnp.int32),
        pltpu.VMEM((_C, 64), _f32),
        pltpu.VMEM((_C, 64), _f32),
        pltpu.VMEM((32, 64), _f32),
        pltpu.VMEM_SHARED((_NP, 64), _f32),
        pltpu.SemaphoreType.DMA,
        pltpu.SemaphoreType.DMA,
    ],
)
def _spmm_sc(ta, tb, srcp, dstp, zsmall, outa, outb,
             s0, d0, s1, d1, rows0, rows1, zb, acc, semg0, semg1):
    c = lax.axis_index("c")
    s = lax.axis_index("s")
    w = c * _NS + s
    base0 = w * _EW
    pltpu.sync_copy(zsmall, zb)

    def load_idx(j, is_, id_):
        b = base0 + j * _C
        pltpu.sync_copy(srcp.at[pl.ds(b, _C)], is_)
        pltpu.sync_copy(dstp.at[pl.ds(b, _C)], id_)

    half = _ECH // 2
    for table, out in ((ta, outa), (tb, outb)):
        @pl.loop(0, _RZ // 32)
        def _(i):
            pltpu.sync_copy(zb, acc.at[pl.ds(s * _RZ + i * 32, 32)])

        plsc.subcore_barrier()

        load_idx(0, s0, d0)
        pltpu.async_copy(table.at[s0], rows0, semg0)

        @pl.loop(0, half)
        def _(g):
            load_idx(2 * g + 1, s1, d1)
            pltpu.async_copy(table.at[s1], rows1, semg1)
            pltpu.make_async_copy(table.at[s0], rows0, semg0).wait()
            pltpu.sync_copy(rows0, acc.at[d0], add=True)

            @pl.when(g < half - 1)
            def _():
                load_idx(2 * g + 2, s0, d0)
                pltpu.async_copy(table.at[s0], rows0, semg0)

            pltpu.make_async_copy(table.at[s1], rows1, semg1).wait()
            pltpu.sync_copy(rows1, acc.at[d1], add=True)

        plsc.subcore_barrier()
        pltpu.sync_copy(acc.at[pl.ds(s * _RZ, _RZ)], out.at[c, s])


@functools.partial(
    pl.kernel, mesh=_mesh,
    compiler_params=pltpu.CompilerParams(use_tc_tiling_on_sc=False),
    out_type=jax.ShapeDtypeStruct((_NC, _NS, _CZ, 8), _f32),
    scratch_types=[
        pltpu.VMEM((_C,), jnp.int32),
        pltpu.VMEM((_C,), jnp.int32),
        pltpu.VMEM((_C, 8), _f32),
        pltpu.VMEM((_CZ, 8), _f32),
        pltpu.VMEM_SHARED((_CP, 8), _f32),
        pltpu.SemaphoreType.DMA,
    ],
)
def _pool_sc(y, rowp, colp, zpool, out, idx_r, idx_c, rows, zbp, accp, sem):
    c = lax.axis_index("c")
    s = lax.axis_index("s")
    w = c * _NS + s
    pltpu.sync_copy(zpool, zbp)
    pltpu.sync_copy(zbp, accp.at[pl.ds(s * _CZ, _CZ)])
    plsc.subcore_barrier()

    @pl.loop(0, _PCH)
    def _(j):
        b = w * _PW + j * _C
        pltpu.sync_copy(rowp.at[pl.ds(b, _C)], idx_r)
        pltpu.sync_copy(colp.at[pl.ds(b, _C)], idx_c)
        pltpu.async_copy(y.at[idx_r], rows, sem).wait()
        pltpu.sync_copy(rows, accp.at[idx_c], add=True)

    plsc.subcore_barrier()
    pltpu.sync_copy(accp.at[pl.ds(s * _CZ, _CZ)], out.at[c, s])


def _prep1(xtp, w1b, dega, degb):
    def body(x_ref, w_ref, da, db, o_ref):
        inv = lax.rsqrt(1.0 + da[...] + db[...])
        o_ref[...] = inv * jnp.dot(x_ref[...], w_ref[...],
                                   preferred_element_type=_f32)

    return pl.pallas_call(
        body,
        grid=(_NP // 128,),
        in_specs=[
            pl.BlockSpec((128, _BS * 16), lambda i: (i, 0)),
            pl.BlockSpec((_BS * 16, 128), lambda i: (0, 0)),
            pl.BlockSpec((128, 1), lambda i: (i, 0)),
            pl.BlockSpec((128, 1), lambda i: (i, 0)),
        ],
        out_specs=pl.BlockSpec((128, 128), lambda i: (i, 0)),
        out_shape=jax.ShapeDtypeStruct((_NP, 128), _f32),
    )(xtp, w1b, dega, degb)


def _prep23(acca, accb, msgp, dega, degb, wbig, btile):
    def body(aa, ab, mp, da, db, w_ref, b_ref, o_msg, o_xr):
        inv = lax.rsqrt(1.0 + da[...] + db[...])
        xr = jnp.maximum(inv * (aa[...] + ab[...] + mp[...]) + b_ref[...], 0.0)
        o_xr[...] = xr
        o_msg[...] = inv * jnp.dot(xr, w_ref[...], preferred_element_type=_f32)

    blk = pl.BlockSpec((128, 128), lambda i: (i, 0))
    return pl.pallas_call(
        body,
        grid=(_NP // 128,),
        in_specs=[
            blk, blk, blk,
            pl.BlockSpec((128, 1), lambda i: (i, 0)),
            pl.BlockSpec((128, 1), lambda i: (i, 0)),
            pl.BlockSpec((128, 128), lambda i: (0, 0)),
            pl.BlockSpec((1, 128), lambda i: (0, 0)),
        ],
        out_specs=[blk, blk],
        out_shape=[jax.ShapeDtypeStruct((_NP, 128), _f32),
                   jax.ShapeDtypeStruct((_NP, 128), _f32)],
    )(acca, accb, msgp, dega, degb, wbig, btile)


def _ykern(acca, accb, msg3, dega, degb, b3t, xr1, xr2, f1, f2, f3):
    def body(aa, ab, m3, da, db, b3, x1, x2, f1r, f2r, f3r, o_ref):
        i = pl.program_id(0)
        inv = lax.rsqrt(1.0 + da[...] + db[...])
        xr3 = jnp.maximum(inv * (aa[...] + ab[...] + m3[...]) + b3[...], 0.0)
        y = (jnp.dot(x1[...], f1r[...], preferred_element_type=_f32)
             + jnp.dot(x2[...], f2r[...], preferred_element_type=_f32)
             + jnp.dot(xr3, f3r[...], preferred_element_type=_f32))
        rowid = i * 128 + lax.broadcasted_iota(jnp.int32, (128, 8), 0)
        colid = lax.broadcasted_iota(jnp.int32, (128, 8), 1)
        o_ref[...] = y + jnp.where((rowid < _N) & (colid == 4), 1.0, 0.0)

    blk = pl.BlockSpec((128, 128), lambda i: (i, 0))
    fblk = pl.BlockSpec((128, 8), lambda i: (0, 0))
    return pl.pallas_call(
        body,
        grid=(_NP // 128,),
        in_specs=[
            blk, blk, blk,
            pl.BlockSpec((128, 1), lambda i: (i, 0)),
            pl.BlockSpec((128, 1), lambda i: (i, 0)),
            pl.BlockSpec((1, 128), lambda i: (0, 0)),
            blk, blk, fblk, fblk, fblk,
        ],
        out_specs=pl.BlockSpec((128, 8), lambda i: (i, 0)),
        out_shape=jax.ShapeDtypeStruct((_NP, 8), _f32),
    )(acca, accb, msg3, dega, degb, b3t, xr1, xr2, f1, f2, f3)


def _head(pa, pb, lin1p, l1b, l2w, l2b, fcb):
    def body(pa_r, pb_r, w1_r, b1_r, w2_r, b2_r, fb_r, o_ref):
        ps = pa_r[...] + pb_r[...]
        colid = lax.broadcasted_iota(jnp.int32, (_CP, 8), 1)
        is_cnt = (colid == 4).astype(_f32)
        cnt = jnp.maximum(jnp.sum(ps * is_cnt, axis=1, keepdims=True), 1.0)
        h = ps / cnt + fb_r[0, 0]
        z = lax.dot_general(h, w1_r[...], (((0,), (0,)), ((), ())),
                            preferred_element_type=_f32)
        z = jnp.maximum(z[0:_BS, :] + b1_r[...], 0.0)
        lg = jnp.dot(z, w2_r[...], preferred_element_type=_f32) + b2_r[...]
        m = jnp.max(lg, axis=1, keepdims=True)
        e = jnp.exp(lg - m)
        o_ref[...] = (lg - m) - jnp.log(jnp.sum(e, axis=1, keepdims=True))

    return pl.pallas_call(
        body,
        out_shape=jax.ShapeDtypeStruct((_BS, 2), _f32),
    )(pa, pb, lin1p, l1b, l2w, l2b, fcb)


def kernel(x, batch, edge_index, row, col, W1, b1, W2, b2, W3, b3,
           fc_w, fc_b, lin1_w, lin1_b, lin2_w, lin2_b):
    eye = jnp.eye(_BS, dtype=_f32)

    xt = jnp.transpose(x, (1, 0, 2)).reshape(_N, _BS * 16)
    xtp = jnp.pad(xt, ((0, _NP - _N), (0, 0)))

    srcp = jnp.pad(edge_index[0], (0, _EP - _E), constant_values=_N)
    dstp = jnp.pad(edge_index[1], (0, _EP - _E), constant_values=_N)
    rowp = jnp.pad(row, (0, _PP - _P), constant_values=_N)
    colp = jnp.pad(col, (0, _PP - _P), constant_values=_NCMT)

    w1b = jnp.kron(eye, W1)
    w2b = jnp.kron(eye, W2)
    w3b = jnp.kron(eye, W3)
    b1t = jnp.tile(b1, _BS)[None, :]
    b2t = jnp.tile(b2, _BS)[None, :]
    b3t = jnp.tile(b3, _BS)[None, :]
    fc3 = fc_w[:, 0].reshape(_H, 3)
    f1, f2, f3 = (jnp.pad(jnp.kron(eye, fc3[:, l:l + 1]), ((0, 0), (0, 4)))
                  for l in range(3))
    lin1p = jnp.pad(lin1_w, ((0, _CP - _NCMT), (0, 0)))

    zsmall = jnp.zeros((32, 64), _f32)
    zflat = jnp.zeros((_RZ,), _f32)
    oflat = jnp.ones((_C,), _f32)
    zpool = jnp.zeros((_CZ, 8), _f32)

    deg = _deg_sc(dstp, zflat, oflat)
    dega = deg[0].reshape(_NP, 1)
    degb = deg[1].reshape(_NP, 1)

    def spmm(msg):
        oa, ob = _spmm_sc(msg[:, :64], msg[:, 64:], srcp, dstp, zsmall)
        return [jnp.concatenate([oa[k].reshape(_NP, 64),
                                 ob[k].reshape(_NP, 64)], axis=1)
                for k in range(_NC)]

    msg1 = _prep1(xtp, w1b, dega, degb)
    a1 = spmm(msg1)
    msg2, xr1 = _prep23(a1[0], a1[1], msg1, dega, degb, w2b, b1t)
    a2 = spmm(msg2)
    msg3, xr2 = _prep23(a2[0], a2[1], msg2, dega, degb, w3b, b2t)
    a3 = spmm(msg3)
    y = _ykern(a3[0], a3[1], msg3, dega, degb, b3t, xr1, xr2,
               f1, f2, f3)
    pool = _pool_sc(y, rowp, colp, zpool)
    return _head(pool[0].reshape(_CP, 8), pool[1].reshape(_CP, 8),
                 lin1p, lin1_b[None, :], lin2_w, lin2_b[None, :],
                 fc_b.reshape(1, 1))
